# baseline (speedup 1.0000x reference)
import functools

import jax
import jax.numpy as jnp
from jax import lax
from jax.experimental import pallas as pl
from jax.experimental.pallas import tpu as pltpu
from jax.experimental.pallas import tpu_sc as plsc

EMBED_DIM = 32
BATCH = 16384
NUM_CORES = 2
NUM_SUBCORES = 16
NUM_WORKERS = NUM_CORES * NUM_SUBCORES
B_PER_W = BATCH // NUM_WORKERS
CHUNK = 128
N_CHUNKS = B_PER_W // CHUNK
LANES = 16


def _sc_kernel(uid_hbm, mid_hbm, ut_hbm, mt_hbm, out_hbm,
               idx_u, idx_m, rows_u, rows_m, out_v, sem):
    wid = lax.axis_index("s") * NUM_CORES + lax.axis_index("c")

    pltpu.sync_copy(uid_hbm.at[wid], idx_u)
    pltpu.sync_copy(mid_hbm.at[wid], idx_m)

    copies = []
    for j in range(N_CHUNKS):
        dst = rows_u.at[pl.ds(j * CHUNK, CHUNK)]
        c = pltpu.make_async_copy(ut_hbm.at[idx_u.at[j]], dst, sem)
        c.start()
        copies.append(c)
        dst = rows_m.at[pl.ds(j * CHUNK, CHUNK)]
        c = pltpu.make_async_copy(mt_hbm.at[idx_m.at[j]], dst, sem)
        c.start()
        copies.append(c)
    for c in copies:
        c.wait()

    lane0 = lax.iota(jnp.int32, LANES) == 0

    def body(r, _):
        u0 = rows_u[r, pl.ds(0, LANES)]
        u1 = rows_u[r, pl.ds(LANES, LANES)]
        m0 = rows_m[r, pl.ds(0, LANES)]
        m1 = rows_m[r, pl.ds(LANES, LANES)]
        t = u0 * m0 + u1 * m1
        s = jnp.broadcast_to(jnp.sum(t), (LANES,))
        plsc.store_scatter(out_v, [jnp.full((LANES,), r, jnp.int32)], s,
                           mask=lane0)
        return 0

    lax.fori_loop(0, B_PER_W, body, 0, unroll=4)

    pltpu.sync_copy(out_v, out_hbm.at[wid])


@jax.jit
def _run(user_id, movie_id, user_table, movie_table):
    uid = user_id.astype(jnp.int32).reshape(NUM_WORKERS, N_CHUNKS, CHUNK)
    mid = movie_id.astype(jnp.int32).reshape(NUM_WORKERS, N_CHUNKS, CHUNK)

    k = functools.partial(
        pl.kernel,
        out_type=jax.ShapeDtypeStruct((NUM_WORKERS, B_PER_W), jnp.float32),
        mesh=plsc.VectorSubcoreMesh(core_axis_name="c", subcore_axis_name="s"),
        compiler_params=pltpu.CompilerParams(
            needs_layout_passes=False, use_tc_tiling_on_sc=False),
        scratch_types=[
            pltpu.VMEM((N_CHUNKS, CHUNK), jnp.int32),
            pltpu.VMEM((N_CHUNKS, CHUNK), jnp.int32),
            pltpu.VMEM((B_PER_W, EMBED_DIM), jnp.float32),
            pltpu.VMEM((B_PER_W, EMBED_DIM), jnp.float32),
            pltpu.VMEM((B_PER_W,), jnp.float32),
            pltpu.SemaphoreType.DMA,
        ],
    )(_sc_kernel)
    out = k(uid, mid, user_table, movie_table)
    return out.reshape(BATCH)


def kernel(user_id, movie_id, user_table, movie_table):
    return _run(user_id, movie_id, user_table, movie_table)
